# baseline (speedup 1.0000x reference)
import jax
import jax.numpy as jnp
from jax.experimental import pallas as pl
from jax.experimental.pallas import tpu as pltpu

N = 4096
D = 64
BR = 256
NB = N // BR


def _body(x_ref, pinv0_ref, phi0_ref, pinv1_ref, phi1_ref,
          w0_ref, w1_ref, k0_ref, k1_ref,
          out_ref, xp_ref, y_ref):
    p = pl.program_id(0)
    i = pl.program_id(1)

    @pl.when(jnp.logical_and(p == 0, i == 0))
    def _():
        xp_ref[...] = jnp.dot(x_ref[...], w0_ref[...],
                              preferred_element_type=jnp.float32)

    @pl.when(jnp.logical_and(p == 2, i == 0))
    def _():
        xp_ref[...] = jnp.dot(x_ref[...], w1_ref[...],
                              preferred_element_type=jnp.float32)

    def stage0(pinv_ref, k_ref):
        t = jnp.dot(pinv_ref[...], xp_ref[...],
                    preferred_element_type=jnp.float32)
        y_ref[pl.ds(i * BR, BR), :] = k_ref[pl.ds(i * BR, BR), :] * t

    def stage1(phi_ref):
        z = jnp.dot(phi_ref[...], y_ref[...],
                    preferred_element_type=jnp.float32)
        out_ref[0, :, :] = jnp.maximum(z, 0.0)

    @pl.when(p == 0)
    def _():
        stage0(pinv0_ref, k0_ref)

    @pl.when(p == 1)
    def _():
        stage1(phi0_ref)

    @pl.when(p == 2)
    def _():
        stage0(pinv1_ref, k1_ref)

    @pl.when(p == 3)
    def _():
        stage1(phi1_ref)


def kernel(x, phi_inv_0, phi_0, phi_inv_1, phi_1, W0, W1, k0, k1):
    def basis_map(active_phase):
        def imap(p, i):
            blk = jnp.where(p < active_phase, 0,
                            jnp.where(p == active_phase, i, NB - 1))
            return (blk, 0)
        return imap

    def out_map(p, i):
        s = p // 2
        stage = p % 2
        return (s, jnp.where(stage == 1, i, 0), 0)

    full = lambda shape: pl.BlockSpec(shape, lambda p, i: (0,) * len(shape))

    return pl.pallas_call(
        _body,
        grid=(4, NB),
        in_specs=[
            full((N, D)),
            pl.BlockSpec((BR, N), basis_map(0)),
            pl.BlockSpec((BR, N), basis_map(1)),
            pl.BlockSpec((BR, N), basis_map(2)),
            pl.BlockSpec((BR, N), basis_map(3)),
            full((D, D)),
            full((D, D)),
            full((N, 1)),
            full((N, 1)),
        ],
        out_specs=pl.BlockSpec((1, BR, D), out_map),
        out_shape=jax.ShapeDtypeStruct((2, N, D), jnp.float32),
        scratch_shapes=[
            pltpu.VMEM((N, D), jnp.float32),
            pltpu.VMEM((N, D), jnp.float32),
        ],
    )(x, phi_inv_0, phi_0, phi_inv_1, phi_1, W0, W1, k0, k1)


# speedup vs baseline: 1.0032x; 1.0032x over previous
import jax
import jax.numpy as jnp
from jax.experimental import pallas as pl
from jax.experimental.pallas import tpu as pltpu

N = 4096
D = 64
BR = 256
NB = N // BR


def _body(x_ref, pinv0_ref, phi0_ref, pinv1_ref, phi1_ref,
          w0_ref, w1_ref, k0_ref, k1_ref,
          out_ref, xp_ref, y_ref):
    p = pl.program_id(0)
    i = pl.program_id(1)

    @pl.when(jnp.logical_and(p == 0, i == 0))
    def _():
        xp_ref[...] = jnp.dot(x_ref[...], w0_ref[...],
                              preferred_element_type=jnp.float32)

    @pl.when(jnp.logical_and(p == 2, i == 0))
    def _():
        xp_ref[...] = jnp.dot(x_ref[...], w1_ref[...],
                              preferred_element_type=jnp.float32)

    def stage0(pinv_ref, k_ref):
        t = jnp.dot(pinv_ref[...].astype(jnp.bfloat16),
                    xp_ref[...].astype(jnp.bfloat16),
                    preferred_element_type=jnp.float32)
        y_ref[pl.ds(i * BR, BR), :] = k_ref[pl.ds(i * BR, BR), :] * t

    def stage1(phi_ref):
        z = jnp.dot(phi_ref[...].astype(jnp.bfloat16),
                    y_ref[...].astype(jnp.bfloat16),
                    preferred_element_type=jnp.float32)
        out_ref[0, :, :] = jnp.maximum(z, 0.0)

    @pl.when(p == 0)
    def _():
        stage0(pinv0_ref, k0_ref)

    @pl.when(p == 1)
    def _():
        stage1(phi0_ref)

    @pl.when(p == 2)
    def _():
        stage0(pinv1_ref, k1_ref)

    @pl.when(p == 3)
    def _():
        stage1(phi1_ref)


def kernel(x, phi_inv_0, phi_0, phi_inv_1, phi_1, W0, W1, k0, k1):
    def basis_map(active_phase):
        def imap(p, i):
            blk = jnp.where(p < active_phase, 0,
                            jnp.where(p == active_phase, i, NB - 1))
            return (blk, 0)
        return imap

    def out_map(p, i):
        s = p // 2
        stage = p % 2
        return (s, jnp.where(stage == 1, i, 0), 0)

    full = lambda shape: pl.BlockSpec(shape, lambda p, i: (0,) * len(shape))

    return pl.pallas_call(
        _body,
        grid=(4, NB),
        in_specs=[
            full((N, D)),
            pl.BlockSpec((BR, N), basis_map(0)),
            pl.BlockSpec((BR, N), basis_map(1)),
            pl.BlockSpec((BR, N), basis_map(2)),
            pl.BlockSpec((BR, N), basis_map(3)),
            full((D, D)),
            full((D, D)),
            full((N, 1)),
            full((N, 1)),
        ],
        out_specs=pl.BlockSpec((1, BR, D), out_map),
        out_shape=jax.ShapeDtypeStruct((2, N, D), jnp.float32),
        scratch_shapes=[
            pltpu.VMEM((N, D), jnp.float32),
            pltpu.VMEM((N, D), jnp.float32),
        ],
    )(x, phi_inv_0, phi_0, phi_inv_1, phi_1, W0, W1, k0, k1)


# speedup vs baseline: 1.0340x; 1.0308x over previous
import jax
import jax.numpy as jnp
from jax import lax
from jax.experimental import pallas as pl
from jax.experimental.pallas import tpu as pltpu

N = 4096
D = 64
BR = 256
NB = N // BR

_DN_NT = (((1,), (1,)), ((), ()))


def _body(x_ref, pinv0_ref, phi0_ref, pinv1_ref, phi1_ref,
          w0_ref, w1_ref, k0_ref, k1_ref,
          out_ref, xpt_ref, yt_ref):
    p = pl.program_id(0)
    i = pl.program_id(1)

    def xpt(w_ref):
        xpt_ref[...] = lax.dot_general(
            w_ref[...], x_ref[...], (((0,), (1,)), ((), ())),
            preferred_element_type=jnp.float32)

    @pl.when(jnp.logical_and(p == 0, i == 0))
    def _():
        xpt(w0_ref)

    @pl.when(jnp.logical_and(p == 2, i == 0))
    def _():
        xpt(w1_ref)

    def stage0(pinv_ref, k_ref):
        t = lax.dot_general(xpt_ref[...], pinv_ref[...], _DN_NT,
                            preferred_element_type=jnp.float32)
        yt_ref[:, pl.ds(i * BR, BR)] = k_ref[:, pl.ds(i * BR, BR)] * t

    def stage1(phi_ref):
        z = lax.dot_general(yt_ref[...], phi_ref[...], _DN_NT,
                            preferred_element_type=jnp.float32)
        out_ref[0, :, :] = jnp.maximum(z, 0.0).T

    @pl.when(p == 0)
    def _():
        stage0(pinv0_ref, k0_ref)

    @pl.when(p == 1)
    def _():
        stage1(phi0_ref)

    @pl.when(p == 2)
    def _():
        stage0(pinv1_ref, k1_ref)

    @pl.when(p == 3)
    def _():
        stage1(phi1_ref)


def kernel(x, phi_inv_0, phi_0, phi_inv_1, phi_1, W0, W1, k0, k1):
    def basis_map(active_phase):
        def imap(p, i):
            blk = jnp.where(p < active_phase, 0,
                            jnp.where(p == active_phase, i, NB - 1))
            return (blk, 0)
        return imap

    def out_map(p, i):
        s = p // 2
        stage = p % 2
        return (s, jnp.where(stage == 1, i, 0), 0)

    full = lambda shape: pl.BlockSpec(shape, lambda p, i: (0,) * len(shape))

    out = pl.pallas_call(
        _body,
        grid=(4, NB),
        in_specs=[
            full((N, D)),
            pl.BlockSpec((BR, N), basis_map(0)),
            pl.BlockSpec((BR, N), basis_map(1)),
            pl.BlockSpec((BR, N), basis_map(2)),
            pl.BlockSpec((BR, N), basis_map(3)),
            full((D, D)),
            full((D, D)),
            full((1, N)),
            full((1, N)),
        ],
        out_specs=pl.BlockSpec((1, BR, D), out_map),
        out_shape=jax.ShapeDtypeStruct((2, N, D), jnp.float32),
        scratch_shapes=[
            pltpu.VMEM((D, N), jnp.float32),
            pltpu.VMEM((D, N), jnp.float32),
        ],
    )(x, phi_inv_0, phi_0, phi_inv_1, phi_1, W0, W1,
      k0.reshape(1, N), k1.reshape(1, N))
    return out


# speedup vs baseline: 1.2363x; 1.1956x over previous
import jax
import jax.numpy as jnp
from jax import lax
from jax.experimental import pallas as pl
from jax.experimental.pallas import tpu as pltpu

N = 4096
D = 64
BR = 256
NB = N // BR

_DN_NT = (((1,), (1,)), ((), ()))


def _stage0_body(x_ref, pinv0_ref, pinv1_ref, w0_ref, w1_ref,
                 k0_ref, k1_ref, yt0_ref, yt1_ref, xpt0_ref, xpt1_ref):
    i = pl.program_id(0)

    @pl.when(i == 0)
    def _():
        xpt0_ref[...] = lax.dot_general(
            w0_ref[...], x_ref[...], (((0,), (1,)), ((), ())),
            preferred_element_type=jnp.float32)
        xpt1_ref[...] = lax.dot_general(
            w1_ref[...], x_ref[...], (((0,), (1,)), ((), ())),
            preferred_element_type=jnp.float32)

    t0 = lax.dot_general(xpt0_ref[...], pinv0_ref[...], _DN_NT,
                         preferred_element_type=jnp.float32)
    yt0_ref[...] = k0_ref[:, pl.ds(i * BR, BR)] * t0
    t1 = lax.dot_general(xpt1_ref[...], pinv1_ref[...], _DN_NT,
                         preferred_element_type=jnp.float32)
    yt1_ref[...] = k1_ref[:, pl.ds(i * BR, BR)] * t1


def _stage1_body(yt0_ref, yt1_ref, phi0_ref, phi1_ref, out_ref):
    z0 = lax.dot_general(yt0_ref[...], phi0_ref[...], _DN_NT,
                         preferred_element_type=jnp.float32)
    out_ref[0, :, :] = jnp.maximum(z0, 0.0).T
    z1 = lax.dot_general(yt1_ref[...], phi1_ref[...], _DN_NT,
                         preferred_element_type=jnp.float32)
    out_ref[1, :, :] = jnp.maximum(z1, 0.0).T


def kernel(x, phi_inv_0, phi_0, phi_inv_1, phi_1, W0, W1, k0, k1):
    full = lambda shape: pl.BlockSpec(shape, lambda i: (0,) * len(shape))
    row_blk = pl.BlockSpec((BR, N), lambda i: (i, 0))

    yt0, yt1 = pl.pallas_call(
        _stage0_body,
        grid=(NB,),
        in_specs=[
            full((N, D)),
            row_blk,
            row_blk,
            full((D, D)),
            full((D, D)),
            full((1, N)),
            full((1, N)),
        ],
        out_specs=[pl.BlockSpec((D, BR), lambda i: (0, i)),
                   pl.BlockSpec((D, BR), lambda i: (0, i))],
        out_shape=[jax.ShapeDtypeStruct((D, N), jnp.float32),
                   jax.ShapeDtypeStruct((D, N), jnp.float32)],
        scratch_shapes=[pltpu.VMEM((D, N), jnp.float32),
                        pltpu.VMEM((D, N), jnp.float32)],
    )(x, phi_inv_0, phi_inv_1, W0, W1, k0.reshape(1, N), k1.reshape(1, N))

    out = pl.pallas_call(
        _stage1_body,
        grid=(NB,),
        in_specs=[
            full((D, N)),
            full((D, N)),
            row_blk,
            row_blk,
        ],
        out_specs=pl.BlockSpec((2, BR, D), lambda i: (0, i, 0)),
        out_shape=jax.ShapeDtypeStruct((2, N, D), jnp.float32),
    )(yt0, yt1, phi_0, phi_1)
    return out


# speedup vs baseline: 1.2457x; 1.0076x over previous
import jax
import jax.numpy as jnp
from jax import lax
from jax.experimental import pallas as pl
from jax.experimental.pallas import tpu as pltpu

N = 4096
D = 64
BR = 256
NB = N // BR

_DN_NT = (((1,), (1,)), ((), ()))


def _body(x_ref, pinv0_ref, pinv1_ref, phi0_ref, phi1_ref,
          w0_ref, w1_ref, k0_ref, k1_ref,
          out_ref, xpt0_ref, xpt1_ref, yt0_ref, yt1_ref):
    p = pl.program_id(0)
    i = pl.program_id(1)

    @pl.when(jnp.logical_and(p == 0, i == 0))
    def _():
        xpt0_ref[...] = lax.dot_general(
            w0_ref[...], x_ref[...], (((0,), (1,)), ((), ())),
            preferred_element_type=jnp.float32)
        xpt1_ref[...] = lax.dot_general(
            w1_ref[...], x_ref[...], (((0,), (1,)), ((), ())),
            preferred_element_type=jnp.float32)

    @pl.when(p == 0)
    def _():
        t0 = lax.dot_general(xpt0_ref[...], pinv0_ref[...], _DN_NT,
                             preferred_element_type=jnp.float32)
        yt0_ref[:, pl.ds(i * BR, BR)] = k0_ref[:, pl.ds(i * BR, BR)] * t0
        t1 = lax.dot_general(xpt1_ref[...], pinv1_ref[...], _DN_NT,
                             preferred_element_type=jnp.float32)
        yt1_ref[:, pl.ds(i * BR, BR)] = k1_ref[:, pl.ds(i * BR, BR)] * t1

    @pl.when(p == 1)
    def _():
        z0 = lax.dot_general(yt0_ref[...], phi0_ref[...], _DN_NT,
                             preferred_element_type=jnp.float32)
        out_ref[0, :, :] = jnp.maximum(z0, 0.0).T
        z1 = lax.dot_general(yt1_ref[...], phi1_ref[...], _DN_NT,
                             preferred_element_type=jnp.float32)
        out_ref[1, :, :] = jnp.maximum(z1, 0.0).T


def kernel(x, phi_inv_0, phi_0, phi_inv_1, phi_1, W0, W1, k0, k1):
    def basis_map(active_phase):
        def imap(p, i):
            blk = jnp.where(p < active_phase, 0,
                            jnp.where(p == active_phase, i, NB - 1))
            return (blk, 0)
        return imap

    full = lambda shape: pl.BlockSpec(shape, lambda p, i: (0,) * len(shape))

    return pl.pallas_call(
        _body,
        grid=(2, NB),
        in_specs=[
            full((N, D)),
            pl.BlockSpec((BR, N), basis_map(0)),
            pl.BlockSpec((BR, N), basis_map(0)),
            pl.BlockSpec((BR, N), basis_map(1)),
            pl.BlockSpec((BR, N), basis_map(1)),
            full((D, D)),
            full((D, D)),
            full((1, N)),
            full((1, N)),
        ],
        out_specs=pl.BlockSpec((2, BR, D),
                               lambda p, i: (0, jnp.where(p == 1, i, 0), 0)),
        out_shape=jax.ShapeDtypeStruct((2, N, D), jnp.float32),
        scratch_shapes=[pltpu.VMEM((D, N), jnp.float32),
                        pltpu.VMEM((D, N), jnp.float32),
                        pltpu.VMEM((D, N), jnp.float32),
                        pltpu.VMEM((D, N), jnp.float32)],
    )(x, phi_inv_0, phi_inv_1, phi_0, phi_1, W0, W1,
      k0.reshape(1, N), k1.reshape(1, N))
